# baseline (device time: 42013 ns/iter reference)
import jax
import jax.numpy as jnp
from jax import lax
from jax.experimental import pallas as pl
from jax.experimental.pallas import tpu as pltpu

M = 1024
D = 1024
QM = M // 4

SEND_Y, RECV_Y, SEND_G, RECV_X, RECV_Z, RECV_D = range(6)
S_Y, S_X, S_Z, S_D = range(4)


def kernel(dy, W):
    def body(dy_ref, w_ref, out_ref, bufs, send_sems, recv_sems):
        x = lax.axis_index("x")
        y = lax.axis_index("y")
        z = lax.axis_index("z")

        barrier = pltpu.get_barrier_semaphore()
        for nbr in ((1 - x, y, z), (x, 1 - y, z), (x, y, 1 - z)):
            pl.semaphore_signal(
                barrier, inc=1, device_id=nbr,
                device_id_type=pl.DeviceIdType.MESH,
            )
        pl.semaphore_wait(barrier, 3)

        q = 2 * x + z
        off = q * QM
        dy_blk = dy_ref[pl.ds(off, QM), :].astype(jnp.bfloat16)
        w_all = w_ref[:, :].astype(jnp.bfloat16)
        p = lax.dot_general(
            dy_blk, w_all, (((1,), (1,)), ((), ())),
            preferred_element_type=jnp.float32,
        )

        bufs[SEND_Y] = p.astype(jnp.bfloat16)
        rdma_y = pltpu.make_async_remote_copy(
            src_ref=bufs.at[SEND_Y],
            dst_ref=bufs.at[RECV_Y],
            send_sem=send_sems.at[S_Y],
            recv_sem=recv_sems.at[S_Y],
            device_id=(x, 1 - y, z),
            device_id_type=pl.DeviceIdType.MESH,
        )
        rdma_y.start()
        rdma_y.wait()

        r = p + bufs[RECV_Y].astype(jnp.float32)
        out_ref[pl.ds(off, QM), :] = r
        bufs[SEND_G] = r.astype(jnp.bfloat16)

        rdma_x = pltpu.make_async_remote_copy(
            src_ref=bufs.at[SEND_G],
            dst_ref=bufs.at[RECV_X],
            send_sem=send_sems.at[S_X],
            recv_sem=recv_sems.at[S_X],
            device_id=(1 - x, y, z),
            device_id_type=pl.DeviceIdType.MESH,
        )
        rdma_z = pltpu.make_async_remote_copy(
            src_ref=bufs.at[SEND_G],
            dst_ref=bufs.at[RECV_Z],
            send_sem=send_sems.at[S_Z],
            recv_sem=recv_sems.at[S_Z],
            device_id=(x, y, 1 - z),
            device_id_type=pl.DeviceIdType.MESH,
        )
        rdma_x.start()
        rdma_z.start()

        rdma_x.wait()
        qx = 2 * (1 - x) + z
        out_ref[pl.ds(qx * QM, QM), :] = bufs[RECV_X].astype(jnp.float32)

        rdma_d = pltpu.make_async_remote_copy(
            src_ref=bufs.at[RECV_X],
            dst_ref=bufs.at[RECV_D],
            send_sem=send_sems.at[S_D],
            recv_sem=recv_sems.at[S_D],
            device_id=(x, y, 1 - z),
            device_id_type=pl.DeviceIdType.MESH,
        )
        rdma_d.start()

        rdma_z.wait()
        qz = 2 * x + (1 - z)
        out_ref[pl.ds(qz * QM, QM), :] = bufs[RECV_Z].astype(jnp.float32)

        rdma_d.wait()
        qd = 2 * (1 - x) + (1 - z)
        out_ref[pl.ds(qd * QM, QM), :] = bufs[RECV_D].astype(jnp.float32)

    return pl.pallas_call(
        body,
        out_shape=jax.ShapeDtypeStruct((M, D), jnp.float32),
        in_specs=[
            pl.BlockSpec(memory_space=pltpu.VMEM),
            pl.BlockSpec(memory_space=pltpu.VMEM),
        ],
        out_specs=pl.BlockSpec(memory_space=pltpu.VMEM),
        scratch_shapes=[
            pltpu.VMEM((6, QM, D), jnp.bfloat16),
            pltpu.SemaphoreType.DMA((4,)),
            pltpu.SemaphoreType.DMA((4,)),
        ],
        compiler_params=pltpu.CompilerParams(collective_id=0),
    )(dy, W)


# device time: 35883 ns/iter; 1.1708x vs baseline; 1.1708x over previous
import jax
import jax.numpy as jnp
from jax import lax
from jax.experimental import pallas as pl
from jax.experimental.pallas import tpu as pltpu

M = 1024
D = 1024
K = 4096
QM = M // 4
NC = 8
CW = D // NC
HR = QM // 2

SEND_Y, RECV_Y, SEND_G, RECV_X, RECV_Z, RECV_D = range(6)
S_Y, S_X, S_Z, S_DZ, S_DX = range(5)


def kernel(dy, W):
    def body(dy_hbm, w_hbm, out_ref, dy_f32, dy_bf, w_f32, bufs,
             dy_sem, w_sems, send_sems, recv_sems):
        x = lax.axis_index("x")
        y = lax.axis_index("y")
        z = lax.axis_index("z")

        barrier = pltpu.get_barrier_semaphore()
        for nbr in ((1 - x, y, z), (x, 1 - y, z), (x, y, 1 - z)):
            pl.semaphore_signal(
                barrier, inc=1, device_id=nbr,
                device_id_type=pl.DeviceIdType.MESH,
            )

        q = 2 * x + z
        off = q * QM

        dy_cp = pltpu.make_async_copy(
            dy_hbm.at[pl.ds(off, QM), :], dy_f32, dy_sem)
        dy_cp.start()

        def w_copy(c):
            return pltpu.make_async_copy(
                w_hbm.at[pl.ds(c * CW, CW), :],
                w_f32.at[c % 2],
                w_sems.at[c % 2],
            )

        w_copy(0).start()
        w_copy(1).start()

        pl.semaphore_wait(barrier, 3)

        dy_cp.wait()
        dy_bf[...] = dy_f32[...].astype(jnp.bfloat16)

        def rdma(src_slot, dst_slot, kind, c, dev, row=None):
            rix = () if row is None else (row,)
            return pltpu.make_async_remote_copy(
                src_ref=bufs.at[(src_slot, c) + rix],
                dst_ref=bufs.at[(dst_slot, c) + rix],
                send_sem=send_sems.at[kind, c],
                recv_sem=recv_sems.at[kind, c],
                device_id=dev,
                device_id_type=pl.DeviceIdType.MESH,
            )

        p_vals, y_rd, x_rd, z_rd, dz_rd, dx_rd = {}, {}, {}, {}, {}, {}

        def stage_gemm_ysend(c):
            w_copy(c).wait()
            wv = w_f32[c % 2].astype(jnp.bfloat16)
            if c + 2 < NC:
                w_copy(c + 2).start()
            p = lax.dot_general(
                dy_bf[...], wv, (((1,), (1,)), ((), ())),
                preferred_element_type=jnp.float32,
            )
            p_vals[c] = p
            bufs[SEND_Y, c] = p.astype(jnp.bfloat16)
            y_rd[c] = rdma(SEND_Y, RECV_Y, S_Y, c, (x, 1 - y, z))
            y_rd[c].start()

        def stage_reduce_gsend(c):
            y_rd[c].wait()
            r = p_vals.pop(c) + bufs[RECV_Y, c].astype(jnp.float32)
            out_ref[pl.ds(off, QM), pl.ds(c * CW, CW)] = r
            bufs[SEND_G, c] = r.astype(jnp.bfloat16)
            x_rd[c] = rdma(SEND_G, RECV_X, S_X, c, (1 - x, y, z))
            x_rd[c].start()
            z_rd[c] = rdma(SEND_G, RECV_Z, S_Z, c, (x, y, 1 - z))
            z_rd[c].start()

        def stage_store_fwd(c):
            x_rd[c].wait()
            qx = 2 * (1 - x) + z
            out_ref[pl.ds(qx * QM, QM), pl.ds(c * CW, CW)] = (
                bufs[RECV_X, c].astype(jnp.float32))
            dz_rd[c] = rdma(RECV_X, RECV_D, S_DZ, c, (x, y, 1 - z),
                            row=pl.ds(0, HR))
            dz_rd[c].start()
            z_rd[c].wait()
            qz = 2 * x + (1 - z)
            out_ref[pl.ds(qz * QM, QM), pl.ds(c * CW, CW)] = (
                bufs[RECV_Z, c].astype(jnp.float32))
            dx_rd[c] = rdma(RECV_Z, RECV_D, S_DX, c, (1 - x, y, z),
                            row=pl.ds(HR, HR))
            dx_rd[c].start()

        def stage_diag_store(c):
            dz_rd[c].wait()
            dx_rd[c].wait()
            qd = 2 * (1 - x) + (1 - z)
            out_ref[pl.ds(qd * QM, QM), pl.ds(c * CW, CW)] = (
                bufs[RECV_D, c].astype(jnp.float32))

        for t in range(NC + 3):
            if t < NC:
                stage_gemm_ysend(t)
            if 1 <= t < NC + 1:
                stage_reduce_gsend(t - 1)
            if 2 <= t < NC + 2:
                stage_store_fwd(t - 2)
            if 3 <= t < NC + 3:
                stage_diag_store(t - 3)

    return pl.pallas_call(
        body,
        out_shape=jax.ShapeDtypeStruct((M, D), jnp.float32),
        in_specs=[
            pl.BlockSpec(memory_space=pl.ANY),
            pl.BlockSpec(memory_space=pl.ANY),
        ],
        out_specs=pl.BlockSpec(memory_space=pltpu.VMEM),
        scratch_shapes=[
            pltpu.VMEM((QM, K), jnp.float32),
            pltpu.VMEM((QM, K), jnp.bfloat16),
            pltpu.VMEM((2, CW, K), jnp.float32),
            pltpu.VMEM((6, NC, QM, CW), jnp.bfloat16),
            pltpu.SemaphoreType.DMA,
            pltpu.SemaphoreType.DMA((2,)),
            pltpu.SemaphoreType.DMA((5, NC)),
            pltpu.SemaphoreType.DMA((5, NC)),
        ],
        compiler_params=pltpu.CompilerParams(collective_id=0),
    )(dy, W)


# device time: 30806 ns/iter; 1.3638x vs baseline; 1.1648x over previous
import jax
import jax.numpy as jnp
from jax import lax
from jax.experimental import pallas as pl
from jax.experimental.pallas import tpu as pltpu

M = 1024
D = 1024
K = 4096
QM = M // 4
NC = 8
CW = D // NC
HR = QM // 2

SEND_Y, RECV_Y = range(2)
S_Y, S_X, S_Z, S_DZ, S_DX = range(5)

LAG_B, LAG_C, LAG_D = 2, 4, 7


def kernel(dy, W):
    def body(dy_hbm, w_hbm, out_ref, dy_f32, dy_bf, w_f32, bufs,
             dy_sem, w_sems, send_sems, recv_sems):
        x = lax.axis_index("x")
        y = lax.axis_index("y")
        z = lax.axis_index("z")

        barrier = pltpu.get_barrier_semaphore()
        for nbr in ((1 - x, y, z), (x, 1 - y, z), (x, y, 1 - z)):
            pl.semaphore_signal(
                barrier, inc=1, device_id=nbr,
                device_id_type=pl.DeviceIdType.MESH,
            )

        q = 2 * x + z
        qx = 2 * (1 - x) + z
        qz = 2 * x + (1 - z)
        off = q * QM

        dy_cp = pltpu.make_async_copy(
            dy_hbm.at[pl.ds(off, QM), :], dy_f32, dy_sem)
        dy_cp.start()

        def w_copy(c):
            return pltpu.make_async_copy(
                w_hbm.at[pl.ds(c * CW, CW), :],
                w_f32.at[c % 2],
                w_sems.at[c % 2],
            )

        w_copy(0).start()
        w_copy(1).start()

        pl.semaphore_wait(barrier, 3)

        dy_cp.wait()
        dy_bf[...] = dy_f32[...].astype(jnp.bfloat16)

        def out_rdma(rows, c, kind, dev):
            sl = out_ref.at[rows, pl.ds(c * CW, CW)]
            return pltpu.make_async_remote_copy(
                src_ref=sl, dst_ref=sl,
                send_sem=send_sems.at[kind, c],
                recv_sem=recv_sems.at[kind, c],
                device_id=dev,
                device_id_type=pl.DeviceIdType.MESH,
            )

        p_vals, y_rd, x_rd, z_rd, dz_rd, dx_rd = {}, {}, {}, {}, {}, {}

        def stage_a(c):
            w_copy(c).wait()
            wv = w_f32[c % 2].astype(jnp.bfloat16)
            if c + 2 < NC:
                w_copy(c + 2).start()
            p = lax.dot_general(
                dy_bf[...], wv, (((1,), (1,)), ((), ())),
                preferred_element_type=jnp.float32,
            )
            p_vals[c] = p
            bufs[SEND_Y, c] = p.astype(jnp.bfloat16)
            y_rd[c] = pltpu.make_async_remote_copy(
                src_ref=bufs.at[SEND_Y, c],
                dst_ref=bufs.at[RECV_Y, c],
                send_sem=send_sems.at[S_Y, c],
                recv_sem=recv_sems.at[S_Y, c],
                device_id=(x, 1 - y, z),
                device_id_type=pl.DeviceIdType.MESH,
            )
            y_rd[c].start()

        def stage_b(c):
            y_rd[c].wait()
            r = p_vals.pop(c) + bufs[RECV_Y, c].astype(jnp.float32)
            out_ref[pl.ds(off, QM), pl.ds(c * CW, CW)] = r.astype(jnp.bfloat16)
            x_rd[c] = out_rdma(pl.ds(off, QM), c, S_X, (1 - x, y, z))
            x_rd[c].start()
            z_rd[c] = out_rdma(pl.ds(off, QM), c, S_Z, (x, y, 1 - z))
            z_rd[c].start()

        def stage_c(c):
            x_rd[c].wait()
            dz_rd[c] = out_rdma(pl.ds(qx * QM, HR), c, S_DZ, (x, y, 1 - z))
            dz_rd[c].start()
            z_rd[c].wait()
            dx_rd[c] = out_rdma(pl.ds(qz * QM + HR, HR), c, S_DX,
                                (1 - x, y, z))
            dx_rd[c].start()

        def stage_d(c):
            dz_rd[c].wait()
            dx_rd[c].wait()

        for t in range(NC + LAG_D):
            if t < NC:
                stage_a(t)
            if LAG_B <= t < NC + LAG_B:
                stage_b(t - LAG_B)
            if LAG_C <= t < NC + LAG_C:
                stage_c(t - LAG_C)
            if LAG_D <= t < NC + LAG_D:
                stage_d(t - LAG_D)

    return pl.pallas_call(
        body,
        out_shape=jax.ShapeDtypeStruct((M, D), jnp.bfloat16),
        in_specs=[
            pl.BlockSpec(memory_space=pl.ANY),
            pl.BlockSpec(memory_space=pl.ANY),
        ],
        out_specs=pl.BlockSpec(memory_space=pltpu.VMEM),
        scratch_shapes=[
            pltpu.VMEM((QM, K), jnp.float32),
            pltpu.VMEM((QM, K), jnp.bfloat16),
            pltpu.VMEM((2, CW, K), jnp.float32),
            pltpu.VMEM((2, NC, QM, CW), jnp.bfloat16),
            pltpu.SemaphoreType.DMA,
            pltpu.SemaphoreType.DMA((2,)),
            pltpu.SemaphoreType.DMA((5, NC)),
            pltpu.SemaphoreType.DMA((5, NC)),
        ],
        compiler_params=pltpu.CompilerParams(collective_id=0),
    )(dy, W)


# device time: 29217 ns/iter; 1.4380x vs baseline; 1.0544x over previous
import jax
import jax.numpy as jnp
from jax import lax
from jax.experimental import pallas as pl
from jax.experimental.pallas import tpu as pltpu

M = 1024
D = 1024
K = 4096
QM = M // 4
NC = 8
CW = D // NC
HR = QM // 2
NG = 4
GW = D // NG

SEND_Y, RECV_Y = range(2)
S_Y, S_X, S_Z, S_DZ, S_DX = range(5)

LAG_B, LAG_C, LAG_D = 4, 6, 8


def kernel(dy, W):
    def body(dy_hbm, w_hbm, out_ref, dy_f32, dy_bf, w_f32, bufs,
             dy_sem, w_sems, send_sems, recv_sems):
        x = lax.axis_index("x")
        y = lax.axis_index("y")
        z = lax.axis_index("z")

        barrier = pltpu.get_barrier_semaphore()
        for nbr in ((1 - x, y, z), (x, 1 - y, z), (x, y, 1 - z)):
            pl.semaphore_signal(
                barrier, inc=1, device_id=nbr,
                device_id_type=pl.DeviceIdType.MESH,
            )

        q = 2 * x + z
        qx = 2 * (1 - x) + z
        qz = 2 * x + (1 - z)
        off = q * QM

        dy_cp = pltpu.make_async_copy(
            dy_hbm.at[pl.ds(off, QM), :], dy_f32, dy_sem)
        dy_cp.start()

        def w_copy(g):
            return pltpu.make_async_copy(
                w_hbm.at[pl.ds(g * GW, GW), :], w_f32.at[g], w_sems.at[g])

        for g in range(NG):
            w_copy(g).start()

        pl.semaphore_wait(barrier, 3)

        dy_cp.wait()
        dy_bf[...] = dy_f32[...].astype(jnp.bfloat16)

        def out_rdma(rows, c, kind, dev):
            sl = out_ref.at[rows, pl.ds(c * CW, CW)]
            return pltpu.make_async_remote_copy(
                src_ref=sl, dst_ref=sl,
                send_sem=send_sems.at[kind, c],
                recv_sem=recv_sems.at[kind, c],
                device_id=dev,
                device_id_type=pl.DeviceIdType.MESH,
            )

        p_vals, y_rd, x_rd, z_rd, dz_rd, dx_rd = {}, {}, {}, {}, {}, {}

        def stage_a(t):
            g = t // 2
            w_copy(g).wait()
            wv = w_f32[g].astype(jnp.bfloat16)
            p = lax.dot_general(
                dy_bf[...], wv, (((1,), (1,)), ((), ())),
                preferred_element_type=jnp.float32,
            )
            for i in (0, 1):
                c = 2 * g + i
                pc = p[:, i * CW:(i + 1) * CW]
                p_vals[c] = pc
                bufs[SEND_Y, c] = pc.astype(jnp.bfloat16)
                y_rd[c] = pltpu.make_async_remote_copy(
                    src_ref=bufs.at[SEND_Y, c],
                    dst_ref=bufs.at[RECV_Y, c],
                    send_sem=send_sems.at[S_Y, c],
                    recv_sem=recv_sems.at[S_Y, c],
                    device_id=(x, 1 - y, z),
                    device_id_type=pl.DeviceIdType.MESH,
                )
                y_rd[c].start()

        def stage_b(c):
            y_rd[c].wait()
            r = p_vals.pop(c) + bufs[RECV_Y, c].astype(jnp.float32)
            out_ref[pl.ds(off, QM), pl.ds(c * CW, CW)] = r.astype(jnp.bfloat16)
            x_rd[c] = out_rdma(pl.ds(off, QM), c, S_X, (1 - x, y, z))
            x_rd[c].start()
            z_rd[c] = out_rdma(pl.ds(off, QM), c, S_Z, (x, y, 1 - z))
            z_rd[c].start()

        def stage_c(c):
            x_rd[c].wait()
            dz_rd[c] = out_rdma(pl.ds(qx * QM, HR), c, S_DZ, (x, y, 1 - z))
            dz_rd[c].start()
            z_rd[c].wait()
            dx_rd[c] = out_rdma(pl.ds(qz * QM + HR, HR), c, S_DX,
                                (1 - x, y, z))
            dx_rd[c].start()

        def stage_d(c):
            dz_rd[c].wait()
            dx_rd[c].wait()

        for t in range(NC + LAG_D):
            if t < NC and t % 2 == 0:
                stage_a(t)
            if LAG_B <= t < NC + LAG_B:
                stage_b(t - LAG_B)
            if LAG_C <= t < NC + LAG_C:
                stage_c(t - LAG_C)
            if LAG_D <= t < NC + LAG_D:
                stage_d(t - LAG_D)

    return pl.pallas_call(
        body,
        out_shape=jax.ShapeDtypeStruct((M, D), jnp.bfloat16),
        in_specs=[
            pl.BlockSpec(memory_space=pl.ANY),
            pl.BlockSpec(memory_space=pl.ANY),
        ],
        out_specs=pl.BlockSpec(memory_space=pltpu.VMEM),
        scratch_shapes=[
            pltpu.VMEM((QM, K), jnp.float32),
            pltpu.VMEM((QM, K), jnp.bfloat16),
            pltpu.VMEM((NG, GW, K), jnp.float32),
            pltpu.VMEM((2, NC, QM, CW), jnp.bfloat16),
            pltpu.SemaphoreType.DMA,
            pltpu.SemaphoreType.DMA((NG,)),
            pltpu.SemaphoreType.DMA((5, NC)),
            pltpu.SemaphoreType.DMA((5, NC)),
        ],
        compiler_params=pltpu.CompilerParams(collective_id=0),
    )(dy, W)
